# baseline (speedup 1.0000x reference)
import functools

import jax
import jax.numpy as jnp
from jax import lax
from jax.experimental import pallas as pl
from jax.experimental.pallas import tpu as pltpu
from jax.experimental.pallas import tpu_sc as plsc

B = 16384
NUM_DENSE = 13
NUM_SPARSE = 26
VOCAB = 100000
D = 64
NF = NUM_SPARSE + 1
NPAIR = NF * (NF - 1) // 2
INT_DIM = D + NPAIR

N_ROWS = NUM_SPARSE * VOCAB
LO_TABLES = 13
LO_ROWS = LO_TABLES * VOCAB
HI_ROWS = N_ROWS - LO_ROWS
HI_HALF = HI_ROWS // 2

GATHER_WINDOW = 128
N_IDX_H = B * LO_TABLES

BB = 256

REPACK_BLK = 10000
RPB_H = REPACK_BLK // 2

SC_W = 400
SC_NCHUNK = HI_ROWS // SC_W


def _repack_body(x_ref, o_ref):
    o_ref[:, :D] = x_ref[:RPB_H]
    o_ref[:, D:] = x_ref[RPB_H:]


def _tc_repack(emb_flat):
    return pl.pallas_call(
        _repack_body,
        grid=(LO_ROWS // REPACK_BLK,),
        in_specs=[pl.BlockSpec((REPACK_BLK, D), lambda i: (i, 0))],
        out_specs=pl.BlockSpec((RPB_H, 2 * D), lambda i: (i, 0)),
        out_shape=jax.ShapeDtypeStruct((LO_ROWS // 2, 2 * D), jnp.float32),
    )(emb_flat)


def _sc_repack(emb_flat):
    mesh = plsc.VectorSubcoreMesh(core_axis_name="core", subcore_axis_name="subcore")

    @functools.partial(
        pl.kernel,
        out_type=jax.ShapeDtypeStruct((HI_HALF, 2 * D), jnp.float32),
        mesh=mesh,
        scratch_types=[pltpu.VMEM((SC_W, D), jnp.float32),
                       pltpu.VMEM((SC_W // 2, 2 * D), jnp.float32)],
    )
    def k(x_hbm, o_hbm, buf, mrg):
        wid = lax.axis_index("subcore") * 2 + lax.axis_index("core")

        @pl.loop(0, (SC_NCHUNK + 31) // 32)
        def _(c):
            cid = wid + 32 * c

            @pl.when(cid < SC_NCHUNK)
            def _():
                src = pl.multiple_of(LO_ROWS + cid * SC_W, 8)
                dst = pl.multiple_of(cid * (SC_W // 2), 8)
                pltpu.sync_copy(x_hbm.at[pl.ds(src, SC_W), :], buf)

                @pl.loop(0, SC_W // 2)
                def _(p):
                    for k16 in range(D // 16):
                        s = 16 * k16
                        mrg[p, pl.ds(s, 16)] = buf[2 * p, pl.ds(s, 16)]
                        mrg[p, pl.ds(D + s, 16)] = buf[2 * p + 1, pl.ds(s, 16)]

                pltpu.sync_copy(mrg, o_hbm.at[pl.ds(dst, SC_W // 2), :])

    return k(emb_flat)


def _sc_gather(packed_lo, idx_lo, packed_hi, idx_hi):
    mesh = plsc.VectorSubcoreMesh(core_axis_name="core", subcore_axis_name="subcore")
    ot = jax.ShapeDtypeStruct((N_IDX_H, 2 * D), jnp.float32)

    @functools.partial(pl.kernel, out_type=(ot, ot), mesh=mesh)
    def k(lo_hbm, il_hbm, hi_hbm, ih_hbm, ol_hbm, oh_hbm):
        def body_lo(i_vmem, o_vmem):
            pltpu.sync_copy(lo_hbm.at[i_vmem.at[0]], o_vmem)

        def body_hi(i_vmem, o_vmem):
            pltpu.sync_copy(hi_hbm.at[i_vmem.at[0]], o_vmem)

        for body, ih, oh in ((body_lo, il_hbm, ol_hbm), (body_hi, ih_hbm, oh_hbm)):
            pltpu.emit_pipeline(
                body,
                grid=(N_IDX_H // GATHER_WINDOW,),
                in_specs=[pl.BlockSpec((1, GATHER_WINDOW), lambda i: (0, i))],
                out_specs=[pl.BlockSpec((GATHER_WINDOW, 2 * D), lambda i: (i, 0))],
                core_axis_name=("core", "subcore"),
                dimension_semantics=(pltpu.PARALLEL,),
            )(ih, oh)

    return k(packed_lo, idx_lo, packed_hi, idx_hi)


def _tc_body(numT_ref, emb_lo_ref, emb_hi_ref, par_ref,
             bw0T, bb0, bw1, bb1, bw2, bb2,
             tw0T, tb0, tw1, tb1, tw2, tb2, tw3, tb3, tw4, tb4,
             out_ref):
    f32 = jnp.float32
    xT = numT_ref[...]
    h = jnp.maximum(jnp.dot(bw0T[...], xT, preferred_element_type=f32) + bb0[...], 0.0)
    h = jnp.maximum(jnp.dot(bw1[...], h, preferred_element_type=f32) + bb1[...], 0.0)
    bot = jnp.maximum(jnp.dot(bw2[...], h, preferred_element_type=f32) + bb2[...], 0.0)

    Ts = [bot]
    for f in range(NUM_SPARSE):
        eref = emb_lo_ref if f < LO_TABLES else emb_hi_ref
        gT = eref[f % LO_TABLES].T
        m = par_ref[f]
        Ts.append(gT[:D] + m * (gT[D:] - gT[:D]))
    T3 = jnp.stack(Ts, axis=0)
    zparts = []
    for i in range(1, NF):
        prod = T3[:i] * T3[i][None]
        zparts.append(jnp.sum(prod, axis=1))
    zcat = jnp.concatenate(zparts, axis=0)
    topT = jnp.concatenate([bot, zcat], axis=0)

    y = jnp.maximum(jnp.dot(tw0T[...], topT, preferred_element_type=f32) + tb0[...], 0.0)
    y = jnp.maximum(jnp.dot(tw1[...], y, preferred_element_type=f32) + tb1[...], 0.0)
    y = jnp.maximum(jnp.dot(tw2[...], y, preferred_element_type=f32) + tb2[...], 0.0)
    y = jnp.maximum(jnp.dot(tw3[...], y, preferred_element_type=f32) + tb3[...], 0.0)
    out_ref[...] = jnp.dot(tw4[...], y, preferred_element_type=f32) + tb4[...]


def _tc_forward(numT, emb_lo, emb_hi, parity, weightsT, interpret=False):
    full = lambda a: pl.BlockSpec(a.shape, lambda b: tuple(0 for _ in a.shape))
    in_specs = [
        pl.BlockSpec((NUM_DENSE, BB), lambda b: (0, b)),
        pl.BlockSpec((LO_TABLES, BB, 2 * D), lambda b: (0, b, 0)),
        pl.BlockSpec((LO_TABLES, BB, 2 * D), lambda b: (0, b, 0)),
        pl.BlockSpec((NUM_SPARSE, 1, BB), lambda b: (0, 0, b)),
    ] + [full(w) for w in weightsT]
    out = pl.pallas_call(
        _tc_body,
        grid=(B // BB,),
        in_specs=in_specs,
        out_specs=pl.BlockSpec((1, BB), lambda b: (0, b)),
        out_shape=jax.ShapeDtypeStruct((1, B), jnp.float32),
        interpret=interpret,
    )(numT, emb_lo, emb_hi, parity, *weightsT)
    return out.reshape(B)


def _index_prep(categorical_input):
    offs = (jnp.arange(NUM_SPARSE, dtype=jnp.int32) * VOCAB)[:, None]
    flat_idx = categorical_input.T.astype(jnp.int32) + offs
    vlo = flat_idx[:LO_TABLES]
    blk_i = vlo // REPACK_BLK
    rem = vlo % REPACK_BLK
    idx_lo = (blk_i * RPB_H + rem % RPB_H).reshape(1, N_IDX_H)
    par_lo = (rem // RPB_H).astype(jnp.float32)
    q = flat_idx[LO_TABLES:] - LO_ROWS
    idx_hi = (q >> 1).reshape(1, N_IDX_H)
    par_hi = (q & 1).astype(jnp.float32)
    parity = jnp.concatenate([par_lo, par_hi], axis=0).reshape(NUM_SPARSE, 1, B)
    return idx_lo, idx_hi, parity


def kernel(numerical_input, categorical_input, emb_tables,
           bw0, bb0, bw1, bb1, bw2, bb2,
           tw0, tb0, tw1, tb1, tw2, tb2, tw3, tb3, tw4, tb4):
    emb_flat = emb_tables.reshape(N_ROWS, D)
    packed_lo = _tc_repack(emb_flat)
    packed_hi = _sc_repack(emb_flat)
    idx_lo, idx_hi, parity = _index_prep(categorical_input)
    g_lo, g_hi = _sc_gather(packed_lo, idx_lo, packed_hi, idx_hi)
    emb_lo = g_lo.reshape(LO_TABLES, B, 2 * D)
    emb_hi = g_hi.reshape(LO_TABLES, B, 2 * D)

    numT = numerical_input.T
    col = lambda v: v.reshape(-1, 1)
    weightsT = [
        bw0.T, col(bb0), bw1.T, col(bb1), bw2.T, col(bb2),
        tw0.T, col(tb0), tw1.T, col(tb1), tw2.T, col(tb2),
        tw3.T, col(tb3), tw4.T, col(tb4),
    ]
    return _tc_forward(numT, emb_lo, emb_hi, parity, weightsT)


# speedup vs baseline: 1.2550x; 1.2550x over previous
import functools

import jax
import jax.numpy as jnp
from jax.experimental import pallas as pl
from jax.experimental.pallas import tpu as pltpu
from jax.experimental.pallas import tpu_sc as plsc

B = 16384
NUM_DENSE = 13
NUM_SPARSE = 26
VOCAB = 100000
D = 64
NF = NUM_SPARSE + 1
NPAIR = NF * (NF - 1) // 2
INT_DIM = D + NPAIR

N_ROWS = NUM_SPARSE * VOCAB
GATHER_WINDOW = 128
N_IDX = B * NUM_SPARSE

BB = 256

REPACK_BLK = 10000
RPB_H = REPACK_BLK // 2


def _repack_body(x_ref, o_ref):
    o_ref[:, :D] = x_ref[:RPB_H]
    o_ref[:, D:] = x_ref[RPB_H:]


def _tc_repack(emb_flat):
    return pl.pallas_call(
        _repack_body,
        grid=(N_ROWS // REPACK_BLK,),
        in_specs=[pl.BlockSpec((REPACK_BLK, D), lambda i: (i, 0))],
        out_specs=pl.BlockSpec((RPB_H, 2 * D), lambda i: (i, 0)),
        out_shape=jax.ShapeDtypeStruct((N_ROWS // 2, 2 * D), jnp.float32),
    )(emb_flat)


def _sc_gather(packed, pair_idx):
    mesh = plsc.VectorSubcoreMesh(core_axis_name="core", subcore_axis_name="subcore")

    @functools.partial(
        pl.kernel,
        out_type=jax.ShapeDtypeStruct((N_IDX, 2 * D), jnp.float32),
        mesh=mesh,
    )
    def k(x_hbm, i_hbm, o_hbm):
        def body(i_vmem, o_vmem):
            pltpu.sync_copy(x_hbm.at[i_vmem.at[0]], o_vmem)

        pltpu.emit_pipeline(
            body,
            grid=(N_IDX // GATHER_WINDOW,),
            in_specs=[pl.BlockSpec((1, GATHER_WINDOW), lambda i: (0, i))],
            out_specs=[pl.BlockSpec((GATHER_WINDOW, 2 * D), lambda i: (i, 0))],
            core_axis_name=("core", "subcore"),
            dimension_semantics=(pltpu.PARALLEL,),
        )(i_hbm, o_hbm)

    return k(packed, pair_idx)


def _tc_body(numT_ref, emb_ref, par_ref,
             bw0T, bb0, bw1, bb1, bw2, bb2,
             tw0T, tb0, tw1, tb1, tw2, tb2, tw3, tb3, tw4, tb4,
             out_ref):
    f32 = jnp.float32
    xT = numT_ref[...]
    h = jnp.maximum(jnp.dot(bw0T[...], xT, preferred_element_type=f32) + bb0[...], 0.0)
    h = jnp.maximum(jnp.dot(bw1[...], h, preferred_element_type=f32) + bb1[...], 0.0)
    bot = jnp.maximum(jnp.dot(bw2[...], h, preferred_element_type=f32) + bb2[...], 0.0)

    Ts = [bot]
    for f in range(NUM_SPARSE):
        gT = emb_ref[f].T
        m = par_ref[f]
        Ts.append(gT[:D] + m * (gT[D:] - gT[:D]))
    T3 = jnp.stack(Ts, axis=0)
    zparts = []
    for i in range(1, NF):
        prod = T3[:i] * T3[i][None]
        zparts.append(jnp.sum(prod, axis=1))
    zcat = jnp.concatenate(zparts, axis=0)
    topT = jnp.concatenate([bot, zcat], axis=0)

    y = jnp.maximum(jnp.dot(tw0T[...], topT, preferred_element_type=f32) + tb0[...], 0.0)
    y = jnp.maximum(jnp.dot(tw1[...], y, preferred_element_type=f32) + tb1[...], 0.0)
    y = jnp.maximum(jnp.dot(tw2[...], y, preferred_element_type=f32) + tb2[...], 0.0)
    y = jnp.maximum(jnp.dot(tw3[...], y, preferred_element_type=f32) + tb3[...], 0.0)
    out_ref[...] = jnp.dot(tw4[...], y, preferred_element_type=f32) + tb4[...]


def _tc_forward(numT, emb_pk, parity, weightsT, interpret=False):
    full = lambda a: pl.BlockSpec(a.shape, lambda b: tuple(0 for _ in a.shape))
    in_specs = [
        pl.BlockSpec((NUM_DENSE, BB), lambda b: (0, b)),
        pl.BlockSpec((NUM_SPARSE, BB, 2 * D), lambda b: (0, b, 0)),
        pl.BlockSpec((NUM_SPARSE, 1, BB), lambda b: (0, 0, b)),
    ] + [full(w) for w in weightsT]
    out = pl.pallas_call(
        _tc_body,
        grid=(B // BB,),
        in_specs=in_specs,
        out_specs=pl.BlockSpec((1, BB), lambda b: (0, b)),
        out_shape=jax.ShapeDtypeStruct((1, B), jnp.float32),
        interpret=interpret,
    )(numT, emb_pk, parity, *weightsT)
    return out.reshape(B)


def _index_prep(categorical_input):
    offs = (jnp.arange(NUM_SPARSE, dtype=jnp.int32) * VOCAB)[:, None]
    flat_idx = categorical_input.T.astype(jnp.int32) + offs
    blk_i = flat_idx // REPACK_BLK
    rem = flat_idx % REPACK_BLK
    pair_idx = (blk_i * RPB_H + rem % RPB_H).reshape(1, N_IDX)
    parity = (rem // RPB_H).astype(jnp.float32).reshape(NUM_SPARSE, 1, B)
    return pair_idx, parity


def kernel(numerical_input, categorical_input, emb_tables,
           bw0, bb0, bw1, bb1, bw2, bb2,
           tw0, tb0, tw1, tb1, tw2, tb2, tw3, tb3, tw4, tb4):
    emb_flat = emb_tables.reshape(N_ROWS, D)
    packed = _tc_repack(emb_flat)
    pair_idx, parity = _index_prep(categorical_input)
    gathered = _sc_gather(packed, pair_idx)
    emb_pk = gathered.reshape(NUM_SPARSE, B, 2 * D)

    numT = numerical_input.T
    col = lambda v: v.reshape(-1, 1)
    weightsT = [
        bw0.T, col(bb0), bw1.T, col(bb1), bw2.T, col(bb2),
        tw0.T, col(tb0), tw1.T, col(tb1), tw2.T, col(tb2),
        tw3.T, col(tb3), tw4.T, col(tb4),
    ]
    return _tc_forward(numT, emb_pk, parity, weightsT)


# speedup vs baseline: 1.2674x; 1.0098x over previous
import functools

import jax
import jax.numpy as jnp
from jax.experimental import pallas as pl
from jax.experimental.pallas import tpu as pltpu
from jax.experimental.pallas import tpu_sc as plsc

B = 16384
NUM_DENSE = 13
NUM_SPARSE = 26
VOCAB = 100000
D = 64
NF = NUM_SPARSE + 1
NPAIR = NF * (NF - 1) // 2
INT_DIM = D + NPAIR

N_ROWS = NUM_SPARSE * VOCAB
GATHER_WINDOW = 128
N_IDX = B * NUM_SPARSE

BB = 256

REPACK_BLK = 20000
RPB_H = REPACK_BLK // 2


def _repack_body(x_ref, o_ref):
    o_ref[:, :D] = x_ref[:RPB_H]
    o_ref[:, D:] = x_ref[RPB_H:]


def _tc_repack(emb_flat):
    return pl.pallas_call(
        _repack_body,
        grid=(N_ROWS // REPACK_BLK,),
        in_specs=[pl.BlockSpec((REPACK_BLK, D), lambda i: (i, 0))],
        out_specs=pl.BlockSpec((RPB_H, 2 * D), lambda i: (i, 0)),
        out_shape=jax.ShapeDtypeStruct((N_ROWS // 2, 2 * D), jnp.float32),
    )(emb_flat)


def _sc_gather(packed, pair_idx):
    mesh = plsc.VectorSubcoreMesh(core_axis_name="core", subcore_axis_name="subcore")

    @functools.partial(
        pl.kernel,
        out_type=jax.ShapeDtypeStruct((N_IDX, 2 * D), jnp.float32),
        mesh=mesh,
    )
    def k(x_hbm, i_hbm, o_hbm):
        def body(i_vmem, o_vmem):
            pltpu.sync_copy(x_hbm.at[i_vmem.at[0]], o_vmem)

        pltpu.emit_pipeline(
            body,
            grid=(N_IDX // GATHER_WINDOW,),
            in_specs=[pl.BlockSpec((1, GATHER_WINDOW), lambda i: (0, i))],
            out_specs=[pl.BlockSpec((GATHER_WINDOW, 2 * D), lambda i: (i, 0))],
            core_axis_name=("core", "subcore"),
            dimension_semantics=(pltpu.PARALLEL,),
        )(i_hbm, o_hbm)

    return k(packed, pair_idx)


def _tc_body(numT_ref, emb_ref, par_ref,
             bw0T, bb0, bw1, bb1, bw2, bb2,
             tw0T, tb0, tw1, tb1, tw2, tb2, tw3, tb3, tw4, tb4,
             out_ref):
    f32 = jnp.float32
    xT = numT_ref[...]
    h = jnp.maximum(jnp.dot(bw0T[...], xT, preferred_element_type=f32) + bb0[...], 0.0)
    h = jnp.maximum(jnp.dot(bw1[...], h, preferred_element_type=f32) + bb1[...], 0.0)
    bot = jnp.maximum(jnp.dot(bw2[...], h, preferred_element_type=f32) + bb2[...], 0.0)

    Ts = [bot]
    for f in range(NUM_SPARSE):
        gT = emb_ref[f].T
        m = par_ref[f]
        Ts.append(gT[:D] + m * (gT[D:] - gT[:D]))
    T3 = jnp.stack(Ts, axis=0)
    zparts = []
    for i in range(1, NF):
        prod = T3[:i] * T3[i][None]
        zparts.append(jnp.sum(prod, axis=1))
    zcat = jnp.concatenate(zparts, axis=0)
    topT = jnp.concatenate([bot, zcat], axis=0)

    y = jnp.maximum(jnp.dot(tw0T[...], topT, preferred_element_type=f32) + tb0[...], 0.0)
    y = jnp.maximum(jnp.dot(tw1[...], y, preferred_element_type=f32) + tb1[...], 0.0)
    y = jnp.maximum(jnp.dot(tw2[...], y, preferred_element_type=f32) + tb2[...], 0.0)
    y = jnp.maximum(jnp.dot(tw3[...], y, preferred_element_type=f32) + tb3[...], 0.0)
    out_ref[...] = jnp.dot(tw4[...], y, preferred_element_type=f32) + tb4[...]


def _tc_forward(numT, emb_pk, parity, weightsT, interpret=False):
    full = lambda a: pl.BlockSpec(a.shape, lambda b: tuple(0 for _ in a.shape))
    in_specs = [
        pl.BlockSpec((NUM_DENSE, BB), lambda b: (0, b)),
        pl.BlockSpec((NUM_SPARSE, BB, 2 * D), lambda b: (0, b, 0)),
        pl.BlockSpec((NUM_SPARSE, 1, BB), lambda b: (0, 0, b)),
    ] + [full(w) for w in weightsT]
    out = pl.pallas_call(
        _tc_body,
        grid=(B // BB,),
        in_specs=in_specs,
        out_specs=pl.BlockSpec((1, BB), lambda b: (0, b)),
        out_shape=jax.ShapeDtypeStruct((1, B), jnp.float32),
        interpret=interpret,
    )(numT, emb_pk, parity, *weightsT)
    return out.reshape(B)


def _index_prep(categorical_input):
    offs = (jnp.arange(NUM_SPARSE, dtype=jnp.int32) * VOCAB)[:, None]
    flat_idx = categorical_input.T.astype(jnp.int32) + offs
    blk_i = flat_idx // REPACK_BLK
    rem = flat_idx % REPACK_BLK
    pair_idx = (blk_i * RPB_H + rem % RPB_H).reshape(1, N_IDX)
    parity = (rem // RPB_H).astype(jnp.float32).reshape(NUM_SPARSE, 1, B)
    return pair_idx, parity


def kernel(numerical_input, categorical_input, emb_tables,
           bw0, bb0, bw1, bb1, bw2, bb2,
           tw0, tb0, tw1, tb1, tw2, tb2, tw3, tb3, tw4, tb4):
    emb_flat = emb_tables.reshape(N_ROWS, D)
    packed = _tc_repack(emb_flat)
    pair_idx, parity = _index_prep(categorical_input)
    gathered = _sc_gather(packed, pair_idx)
    emb_pk = gathered.reshape(NUM_SPARSE, B, 2 * D)

    numT = numerical_input.T
    col = lambda v: v.reshape(-1, 1)
    weightsT = [
        bw0.T, col(bb0), bw1.T, col(bb1), bw2.T, col(bb2),
        tw0.T, col(tb0), tw1.T, col(tb1), tw2.T, col(tb2),
        tw3.T, col(tb3), tw4.T, col(tb4),
    ]
    return _tc_forward(numT, emb_pk, parity, weightsT)
